# baseline (speedup 1.0000x reference)
import jax
import jax.numpy as jnp
from jax import lax
from jax.experimental import pallas as pl
from jax.experimental.pallas import tpu as pltpu
from jax.experimental.pallas import tpu_sc as plsc

NV = 100000
D = 64
B = 1024
L = 20
E = B * L
NC, NS = 2, 16
NW = NC * NS
EPW = E // NW
GJ = EPW // 128
PRES_PAD = 100096
TILE_P = PRES_PAD // NS
RB = 2000
RG = NV // RB
CB = 2048
CG = E // CB
NF = float(NV)


def _sc_body(ug_idx, pg_idx, ng_idx, u1_idx,
             item_cnt, user_cnt,
             g_u_int, g_u_pop, g_ip_int, g_ip_pop, g_in_int, g_in_pop,
             g_u1_int, g_u1_pop,
             pres_sp, idx_v, uidx_v, pidx_v, nidx_v, u1idx_v,
             ones_v, zer_v, r0, r1, r2, r3, r4, r5, sem):
  c = lax.axis_index("c")
  s = lax.axis_index("s")
  wid = c * NS + s

  def _zb(i, carry):
    zer_v[pl.ds(i * 16, 16)] = jnp.zeros((16,), jnp.float32)
    return carry
  lax.fori_loop(0, TILE_P // 16, _zb, 0)
  for i in range(8):
    ones_v[pl.ds(i * 16, 16)] = jnp.full((16,), 1.0, jnp.float32)

  pltpu.sync_copy(zer_v, pres_sp.at[pl.ds(s * TILE_P, TILE_P)])
  plsc.subcore_barrier()

  @pl.when(c == 0)
  def _():
    pltpu.sync_copy(pg_idx.at[2 * s], idx_v.at[pl.ds(0, GJ)])
    pltpu.sync_copy(pg_idx.at[2 * s + 1], idx_v.at[pl.ds(GJ, GJ)])
    pltpu.sync_copy(ng_idx.at[2 * s], idx_v.at[pl.ds(2 * GJ, GJ)])
    pltpu.sync_copy(ng_idx.at[2 * s + 1], idx_v.at[pl.ds(3 * GJ, GJ)])
    for j in range(4 * GJ):
      pltpu.sync_copy(ones_v, pres_sp.at[idx_v.at[j]], add=True)

  @pl.when(c == 1)
  def _():
    pltpu.sync_copy(ug_idx.at[2 * s], idx_v.at[pl.ds(0, GJ)])
    pltpu.sync_copy(ug_idx.at[2 * s + 1], idx_v.at[pl.ds(GJ, GJ)])
    for j in range(2 * GJ):
      pltpu.sync_copy(ones_v, pres_sp.at[idx_v.at[j]], add=True)

  plsc.subcore_barrier()

  pltpu.sync_copy(pres_sp.at[pl.ds(s * TILE_P, TILE_P)], zer_v)

  @pl.when(c == 0)
  def _():
    pltpu.sync_copy(zer_v, item_cnt.at[pl.ds(s * TILE_P, TILE_P)])

  @pl.when(c == 1)
  def _():
    pltpu.sync_copy(zer_v, user_cnt.at[pl.ds(s * TILE_P, TILE_P)])

  pltpu.sync_copy(ug_idx.at[wid], uidx_v)
  pltpu.sync_copy(pg_idx.at[wid], pidx_v)
  pltpu.sync_copy(ng_idx.at[wid], nidx_v)
  for j in range(0):
    d0 = pltpu.async_copy(u_int_w.at[uidx_v.at[j]], r0, sem)
    d1 = pltpu.async_copy(u_pop_w.at[uidx_v.at[j]], r1, sem)
    d2 = pltpu.async_copy(i_int_w.at[pidx_v.at[j]], r2, sem)
    d3 = pltpu.async_copy(i_pop_w.at[pidx_v.at[j]], r3, sem)
    d4 = pltpu.async_copy(i_int_w.at[nidx_v.at[j]], r4, sem)
    d5 = pltpu.async_copy(i_pop_w.at[nidx_v.at[j]], r5, sem)
    d0.wait(); d1.wait(); d2.wait(); d3.wait(); d4.wait(); d5.wait()
    base = wid * EPW + j * 128
    pltpu.sync_copy(r0, g_u_int.at[pl.ds(base, 128)])
    pltpu.sync_copy(r1, g_u_pop.at[pl.ds(base, 128)])
    pltpu.sync_copy(r2, g_ip_int.at[pl.ds(base, 128)])
    pltpu.sync_copy(r3, g_ip_pop.at[pl.ds(base, 128)])
    pltpu.sync_copy(r4, g_in_int.at[pl.ds(base, 128)])
    pltpu.sync_copy(r5, g_in_pop.at[pl.ds(base, 128)])

  pltpu.sync_copy(u1_idx.at[wid], u1idx_v)


def _sc_gather_scatter(u_int_w, u_pop_w, i_int_w, i_pop_w,
                       ug_idx, pg_idx, ng_idx, u1_idx):
  f32 = jnp.float32
  out_type = [
      jax.ShapeDtypeStruct((PRES_PAD,), f32),
      jax.ShapeDtypeStruct((PRES_PAD,), f32),
      jax.ShapeDtypeStruct((E, D), f32),
      jax.ShapeDtypeStruct((E, D), f32),
      jax.ShapeDtypeStruct((E, D), f32),
      jax.ShapeDtypeStruct((E, D), f32),
      jax.ShapeDtypeStruct((E, D), f32),
      jax.ShapeDtypeStruct((E, D), f32),
      jax.ShapeDtypeStruct((B, D), f32),
      jax.ShapeDtypeStruct((B, D), f32),
  ]
  scratch = [
      pltpu.VMEM_SHARED((PRES_PAD,), f32),
      pltpu.VMEM((20, 128), jnp.int32),
      pltpu.VMEM((GJ, 128), jnp.int32),
      pltpu.VMEM((GJ, 128), jnp.int32),
      pltpu.VMEM((GJ, 128), jnp.int32),
      pltpu.VMEM((32,), jnp.int32),
      pltpu.VMEM((128,), f32),
      pltpu.VMEM((TILE_P,), f32),
      pltpu.VMEM((128, D), f32),
      pltpu.VMEM((128, D), f32),
      pltpu.VMEM((128, D), f32),
      pltpu.VMEM((128, D), f32),
      pltpu.VMEM((128, D), f32),
      pltpu.VMEM((128, D), f32),
      pltpu.SemaphoreType.DMA,
  ]
  fn = pl.kernel(
      _sc_body,
      out_type=out_type,
      mesh=plsc.VectorSubcoreMesh(core_axis_name="c", subcore_axis_name="s"),
      scratch_types=scratch,
      compiler_params=pltpu.CompilerParams(use_tc_tiling_on_sc=False),
  )
  return fn(ug_idx, pg_idx, ng_idx, u1_idx)


def _sweep_body(ii_ref, ip_ref, ui_ref, up_ref, icnt_ref, ucnt_ref,
                gi_ref, gp_ref, sv_ref, scal_ref):
  k = pl.program_id(0)

  @pl.when(k == 0)
  def _():
    gi_ref[...] = jnp.zeros((D, D), jnp.float32)
    gp_ref[...] = jnp.zeros((D, D), jnp.float32)
    sv_ref[...] = jnp.zeros((2, D), jnp.float32)
    scal_ref[0] = 0.0
    scal_ref[1] = 0.0
    scal_ref[2] = 0.0
    scal_ref[3] = 0.0

  a = ii_ref[...]
  b = ip_ref[...]
  cu = ui_ref[...]
  du = up_ref[...]
  dn = (((0,), (0,)), ((), ()))
  gi_ref[...] += lax.dot_general(a, a, dn, preferred_element_type=jnp.float32)
  gp_ref[...] += lax.dot_general(b, b, dn, preferred_element_type=jnp.float32)
  sv_ref[0:1, :] += jnp.sum(a, axis=0, keepdims=True)
  sv_ref[1:2, :] += jnp.sum(b, axis=0, keepdims=True)

  im_f = (icnt_ref[0] > 0.0).astype(jnp.float32)
  dif = a - b
  scal_ref[0] += jnp.sum(dif * dif * im_f)
  scal_ref[1] += jnp.sum(im_f)
  um_f = (ucnt_ref[0] > 0.0).astype(jnp.float32)
  difu = cu - du
  scal_ref[2] += jnp.sum(difu * difu * um_f)
  scal_ref[3] += jnp.sum(um_f)


def _sweep(i_int_w, i_pop_w, u_int_w, u_pop_w, icnt, ucnt):
  f32 = jnp.float32
  return pl.pallas_call(
      _sweep_body,
      grid=(RG,),
      in_specs=[
          pl.BlockSpec((RB, D), lambda k: (k, 0)),
          pl.BlockSpec((RB, D), lambda k: (k, 0)),
          pl.BlockSpec((RB, D), lambda k: (k, 0)),
          pl.BlockSpec((RB, D), lambda k: (k, 0)),
          pl.BlockSpec((1, RB, 1), lambda k: (k, 0, 0)),
          pl.BlockSpec((1, RB, 1), lambda k: (k, 0, 0)),
      ],
      out_specs=[
          pl.BlockSpec((D, D), lambda k: (0, 0)),
          pl.BlockSpec((D, D), lambda k: (0, 0)),
          pl.BlockSpec((2, D), lambda k: (0, 0)),
          pl.BlockSpec(memory_space=pltpu.SMEM),
      ],
      out_shape=[
          jax.ShapeDtypeStruct((D, D), f32),
          jax.ShapeDtypeStruct((D, D), f32),
          jax.ShapeDtypeStruct((2, D), f32),
          jax.ShapeDtypeStruct((4,), f32),
      ],
      compiler_params=pltpu.CompilerParams(
          dimension_semantics=("arbitrary",)),
  )(i_int_w, i_pop_w, u_int_w, u_pop_w, icnt, ucnt)


def _nls(x):
  return jnp.maximum(-x, 0.0) + jnp.log(1.0 + jnp.exp(-jnp.abs(x)))


def _fin_body(gui_ref, gup_ref, gpi_ref, gpp_ref, gni_ref, gnp_ref,
              m_ref, u1i_ref, u1p_ref, gi_ref, gp_ref, sv_ref, scal_ref,
              out_ref, acc_ref):
  k = pl.program_id(0)

  @pl.when(k == 0)
  def _():
    acc_ref[0] = 0.0
    acc_ref[1] = 0.0
    acc_ref[2] = 0.0

  dn = (((1,), (0,)), ((), ()))
  ones8 = jnp.ones((D, 8), jnp.float32)
  ui = gui_ref[...]
  up = gup_ref[...]
  zi = ui * (gpi_ref[...] - gni_ref[...])
  zp = up * (gpp_ref[...] - gnp_ref[...])
  xi = lax.dot_general(zi, ones8, dn, preferred_element_type=jnp.float32)
  xp = lax.dot_general(zp, ones8, dn, preferred_element_type=jnp.float32)
  m = m_ref[0]
  acc_ref[0] += jnp.sum(m * _nls(xi))
  acc_ref[1] += jnp.sum(m * _nls(-xp) + (1.0 - m) * _nls(xp))
  acc_ref[2] += jnp.sum(_nls(xi + xp))

  @pl.when(k == CG - 1)
  def _():
    def klstat(U, G, srow):
      t = lax.dot_general(U, G, dn, preferred_element_type=jnp.float32)
      q8 = lax.dot_general(t * U, ones8, dn,
                           preferred_element_type=jnp.float32)
      sv8 = lax.dot_general(U * srow, ones8, dn,
                            preferred_element_type=jnp.float32)
      mean = sv8 / NF
      var = (q8 - NF * mean * mean) / (NF - 1.0)
      std = jnp.sqrt(var) + 1e-8
      kl = -jnp.log(std) + (std * std + mean * mean) * 0.5 - 0.5
      return jnp.sum(kl) / (8.0 * B)

    kli = klstat(u1i_ref[...], gi_ref[...], sv_ref[0:1, :])
    klp = klstat(u1p_ref[...], gp_ref[...], sv_ref[1:2, :])
    disc = (scal_ref[0] / (scal_ref[1] * float(D))
            + scal_ref[2] / (scal_ref[3] * float(D)))
    inv = 1.0 / (8.0 * float(E))
    out_ref[0] = acc_ref[0] * inv
    out_ref[1] = acc_ref[1] * inv
    out_ref[2] = acc_ref[2] * inv
    out_ref[3] = disc
    out_ref[4] = kli
    out_ref[5] = klp
    out_ref[6] = 0.0
    out_ref[7] = 0.0


def _finalize(g_u_int, g_u_pop, g_ip_int, g_ip_pop, g_in_int, g_in_pop,
              mask3, g_u1_int, g_u1_pop, gi, gp, sv, scal):
  f32 = jnp.float32
  blk = lambda: pl.BlockSpec((CB, D), lambda k: (k, 0))
  full2 = lambda r, c: pl.BlockSpec((r, c), lambda k: (0, 0))
  return pl.pallas_call(
      _fin_body,
      grid=(CG,),
      in_specs=[
          blk(), blk(), blk(), blk(), blk(), blk(),
          pl.BlockSpec((1, CB, 1), lambda k: (k, 0, 0)),
          full2(B, D), full2(B, D), full2(D, D), full2(D, D), full2(2, D),
          pl.BlockSpec(memory_space=pltpu.SMEM),
      ],
      out_specs=pl.BlockSpec(memory_space=pltpu.SMEM),
      out_shape=jax.ShapeDtypeStruct((8,), f32),
      scratch_shapes=[pltpu.SMEM((4,), f32)],
      compiler_params=pltpu.CompilerParams(
          dimension_semantics=("arbitrary",)),
  )(g_u_int, g_u_pop, g_ip_int, g_ip_pop, g_in_int, g_in_pop,
    mask3, g_u1_int, g_u1_pop, gi, gp, sv, scal)


def kernel(user, item_p, item_n, mask,
           users_int_w, users_pop_w, items_int_w, items_pop_w):
  i32 = jnp.int32
  user = user.astype(i32)
  item_p = item_p.astype(i32)
  item_n = item_n.astype(i32)

  ug_idx = user.reshape(NW, GJ, 128)
  pg_idx = item_p.reshape(NW, GJ, 128)
  ng_idx = item_n.reshape(NW, GJ, 128)
  u1_idx = user[:, 1].reshape(NW, 32)
  mask3 = mask.reshape(-1).astype(jnp.float32).reshape(CG, CB, 1)

  (item_cnt, user_cnt, g_u_int, g_u_pop, g_ip_int, g_ip_pop,
   g_in_int, g_in_pop, g_u1_int, g_u1_pop) = _sc_gather_scatter(
      users_int_w, users_pop_w, items_int_w, items_pop_w,
      ug_idx, pg_idx, ng_idx, u1_idx)

  icnt3 = item_cnt[:NV].reshape(RG, RB, 1)
  ucnt3 = user_cnt[:NV].reshape(RG, RB, 1)
  gi, gp, sv, scal = _sweep(items_int_w, items_pop_w,
                            users_int_w, users_pop_w, icnt3, ucnt3)

  out = _finalize(g_u_int, g_u_pop, g_ip_int, g_ip_pop, g_in_int, g_in_pop,
                  mask3, g_u1_int, g_u1_pop, gi, gp, sv, scal)
  return (out[0], out[1], out[2], out[3], out[4], out[5])


# speedup vs baseline: 1.4464x; 1.4464x over previous
import jax
import jax.numpy as jnp
from jax import lax
from jax.experimental import pallas as pl
from jax.experimental.pallas import tpu as pltpu
from jax.experimental.pallas import tpu_sc as plsc

NV = 100000
D = 64
DP = 128
B = 1024
L = 20
E = B * L
NC, NS = 2, 16
NW = NC * NS
EPW = E // NW
GJ = EPW // 128
PRES_PAD = 100096
TILE_P = PRES_PAD // NS
RB = 2000
RG = NV // RB
CB = 2048
CG = E // CB
NF = float(NV)


def _sc_body(tu, ti, ug_idx, pg_idx, ng_idx, u1p_idx,
             item_cnt, user_cnt, g_u, g_ip, g_in, g_u1,
             pres_sp, idx_v, uidx_v, pidx_v, nidx_v, u1idx_v,
             ones_v, zer_v, r_u, r_p, r_n, sem):
  c = lax.axis_index("c")
  s = lax.axis_index("s")
  wid = c * NS + s

  def _zb(i, carry):
    zer_v[pl.ds(i * 16, 16)] = jnp.zeros((16,), jnp.float32)
    return carry
  lax.fori_loop(0, TILE_P // 16, _zb, 0)
  for i in range(8):
    ones_v[pl.ds(i * 16, 16)] = jnp.full((16,), 1.0, jnp.float32)

  pltpu.sync_copy(zer_v, pres_sp.at[pl.ds(s * TILE_P, TILE_P)])
  plsc.subcore_barrier()

  @pl.when(c == 0)
  def _():
    pltpu.sync_copy(pg_idx.at[2 * s], idx_v.at[pl.ds(0, 8)])
    pltpu.sync_copy(pg_idx.at[2 * s + 1], idx_v.at[pl.ds(8, 8)])
    pltpu.sync_copy(ng_idx.at[2 * s], idx_v.at[pl.ds(16, 8)])
    pltpu.sync_copy(ng_idx.at[2 * s + 1], idx_v.at[pl.ds(24, 8)])
    ds = []
    for blk in range(4):
      for j in range(GJ):
        ds.append(pltpu.async_copy(
            ones_v, pres_sp.at[idx_v.at[8 * blk + j]], sem, add=True))
    for d in ds:
      d.wait()

  @pl.when(c == 1)
  def _():
    pltpu.sync_copy(ug_idx.at[2 * s], idx_v.at[pl.ds(0, 8)])
    pltpu.sync_copy(ug_idx.at[2 * s + 1], idx_v.at[pl.ds(8, 8)])
    ds = []
    for blk in range(2):
      for j in range(GJ):
        ds.append(pltpu.async_copy(
            ones_v, pres_sp.at[idx_v.at[8 * blk + j]], sem, add=True))
    for d in ds:
      d.wait()

  plsc.subcore_barrier()

  pltpu.sync_copy(pres_sp.at[pl.ds(s * TILE_P, TILE_P)], zer_v)

  @pl.when(c == 0)
  def _():
    pltpu.sync_copy(zer_v, item_cnt.at[pl.ds(s * TILE_P, TILE_P)])

  @pl.when(c == 1)
  def _():
    pltpu.sync_copy(zer_v, user_cnt.at[pl.ds(s * TILE_P, TILE_P)])

  pltpu.sync_copy(ug_idx.at[wid], uidx_v)
  pltpu.sync_copy(pg_idx.at[wid], pidx_v)
  pltpu.sync_copy(ng_idx.at[wid], nidx_v)
  for j in range(GJ):
    d0 = pltpu.async_copy(tu.at[uidx_v.at[j]], r_u, sem)
    d1 = pltpu.async_copy(ti.at[pidx_v.at[j]], r_p, sem)
    d2 = pltpu.async_copy(ti.at[nidx_v.at[j]], r_n, sem)
    d0.wait(); d1.wait(); d2.wait()
    base = wid * EPW + j * 128
    pltpu.sync_copy(r_u, g_u.at[pl.ds(base, 128)])
    pltpu.sync_copy(r_p, g_ip.at[pl.ds(base, 128)])
    pltpu.sync_copy(r_n, g_in.at[pl.ds(base, 128)])

  @pl.when(wid < 8)
  def _():
    pltpu.sync_copy(u1p_idx.at[wid], u1idx_v)
    d = pltpu.async_copy(tu.at[u1idx_v.at[0]], r_u, sem)
    d.wait()
    pltpu.sync_copy(r_u, g_u1.at[pl.ds(wid * 128, 128)])


def _sc_gather_scatter(tu, ti, ug_idx, pg_idx, ng_idx, u1p_idx):
  f32 = jnp.float32
  out_type = [
      jax.ShapeDtypeStruct((PRES_PAD,), f32),
      jax.ShapeDtypeStruct((PRES_PAD,), f32),
      jax.ShapeDtypeStruct((E, DP), f32),
      jax.ShapeDtypeStruct((E, DP), f32),
      jax.ShapeDtypeStruct((E, DP), f32),
      jax.ShapeDtypeStruct((B, DP), f32),
  ]
  scratch = [
      pltpu.VMEM_SHARED((PRES_PAD,), f32),
      pltpu.VMEM((32, 128), jnp.int32),
      pltpu.VMEM((8, 128), jnp.int32),
      pltpu.VMEM((8, 128), jnp.int32),
      pltpu.VMEM((8, 128), jnp.int32),
      pltpu.VMEM((8, 128), jnp.int32),
      pltpu.VMEM((128,), f32),
      pltpu.VMEM((TILE_P,), f32),
      pltpu.VMEM((128, DP), f32),
      pltpu.VMEM((128, DP), f32),
      pltpu.VMEM((128, DP), f32),
      pltpu.SemaphoreType.DMA,
  ]
  fn = pl.kernel(
      _sc_body,
      out_type=out_type,
      mesh=plsc.VectorSubcoreMesh(core_axis_name="c", subcore_axis_name="s"),
      scratch_types=scratch,
      compiler_params=pltpu.CompilerParams(use_tc_tiling_on_sc=True),
  )
  return fn(tu, ti, ug_idx, pg_idx, ng_idx, u1p_idx)


def _ones8(rows, cols):
  return jnp.ones((rows, cols), jnp.float32)


def _sweep_body(ii_ref, ip_ref, ui_ref, up_ref,
                gbig_ref, sv_ref, rsqi_ref, rsqu_ref, tu_ref, ti_ref):
  k = pl.program_id(0)

  @pl.when(k == 0)
  def _():
    gbig_ref[...] = jnp.zeros((DP, DP), jnp.float32)
    sv_ref[...] = jnp.zeros((1, DP), jnp.float32)

  a = ii_ref[...]
  b = ip_ref[...]
  cu = ui_ref[...]
  du = up_ref[...]
  ti = jnp.concatenate([a, b], axis=1)
  tu = jnp.concatenate([cu, du], axis=1)
  ti_ref[...] = ti
  tu_ref[...] = tu
  gbig_ref[...] += lax.dot_general(
      ti, ti, (((0,), (0,)), ((), ())), preferred_element_type=jnp.float32)
  sv_ref[...] += jnp.sum(ti, axis=0, keepdims=True)

  on8 = _ones8(8, D)
  dnT = (((1,), (1,)), ((), ()))
  dif = a - b
  rq = lax.dot_general(on8, dif * dif, dnT,
                       preferred_element_type=jnp.float32)
  rsqi_ref[...] = rq[0:1].reshape(1, 1, RB)
  difu = cu - du
  rqu = lax.dot_general(on8, difu * difu, dnT,
                        preferred_element_type=jnp.float32)
  rsqu_ref[...] = rqu[0:1].reshape(1, 1, RB)


def _sweep(i_int_w, i_pop_w, u_int_w, u_pop_w):
  f32 = jnp.float32
  return pl.pallas_call(
      _sweep_body,
      grid=(RG,),
      in_specs=[
          pl.BlockSpec((RB, D), lambda k: (k, 0)),
          pl.BlockSpec((RB, D), lambda k: (k, 0)),
          pl.BlockSpec((RB, D), lambda k: (k, 0)),
          pl.BlockSpec((RB, D), lambda k: (k, 0)),
      ],
      out_specs=[
          pl.BlockSpec((DP, DP), lambda k: (0, 0)),
          pl.BlockSpec((1, DP), lambda k: (0, 0)),
          pl.BlockSpec((1, 1, RB), lambda k: (k, 0, 0)),
          pl.BlockSpec((1, 1, RB), lambda k: (k, 0, 0)),
          pl.BlockSpec((RB, DP), lambda k: (k, 0)),
          pl.BlockSpec((RB, DP), lambda k: (k, 0)),
      ],
      out_shape=[
          jax.ShapeDtypeStruct((DP, DP), f32),
          jax.ShapeDtypeStruct((1, DP), f32),
          jax.ShapeDtypeStruct((RG, 1, RB), f32),
          jax.ShapeDtypeStruct((RG, 1, RB), f32),
          jax.ShapeDtypeStruct((NV, DP), f32),
          jax.ShapeDtypeStruct((NV, DP), f32),
      ],
      compiler_params=pltpu.CompilerParams(
          dimension_semantics=("arbitrary",)),
  )(i_int_w, i_pop_w, u_int_w, u_pop_w)


def _nls(x):
  return jnp.maximum(-x, 0.0) + jnp.log(1.0 + jnp.exp(-jnp.abs(x)))


def _half_ones():
  r = lax.broadcasted_iota(jnp.int32, (16, DP), 0)
  cidx = lax.broadcasted_iota(jnp.int32, (16, DP), 1)
  left = jnp.logical_and(r < 8, cidx < D)
  right = jnp.logical_and(r >= 8, cidx >= D)
  return jnp.logical_or(left, right).astype(jnp.float32)


def _fin_body(gu_ref, gip_ref, gin_ref, m_ref, gu1_ref, gbig_ref, sv_ref,
              rsqi_ref, rsqu_ref, icnt_ref, ucnt_ref,
              out_ref, acc_ref):
  k = pl.program_id(0)
  dnT = (((1,), (1,)), ((), ()))

  @pl.when(k == 0)
  def _():
    acc_ref[0] = 0.0
    acc_ref[1] = 0.0
    acc_ref[2] = 0.0

  w16 = _half_ones()
  z = gu_ref[...] * (gip_ref[...] - gin_ref[...])
  xq = lax.dot_general(w16, z, dnT, preferred_element_type=jnp.float32)
  xi = xq[0:8]
  xp = xq[8:16]
  m = m_ref[0]
  acc_ref[0] += jnp.sum(m * _nls(xi))
  acc_ref[1] += jnp.sum(m * _nls(-xp) + (1.0 - m) * _nls(xp))
  acc_ref[2] += jnp.sum(_nls(xi + xp))

  @pl.when(k == CG - 1)
  def _():
    u1 = gu1_ref[...]
    rr = lax.broadcasted_iota(jnp.int32, (DP, DP), 0)
    cc = lax.broadcasted_iota(jnp.int32, (DP, DP), 1)
    bm = ((rr < D) == (cc < D)).astype(jnp.float32)
    gd = gbig_ref[...] * bm
    t = lax.dot_general(u1, gd, (((1,), (0,)), ((), ())),
                        preferred_element_type=jnp.float32)
    q16 = lax.dot_general(w16, t * u1, dnT,
                          preferred_element_type=jnp.float32)
    s16 = lax.dot_general(w16, u1 * sv_ref[...], dnT,
                          preferred_element_type=jnp.float32)
    mean = s16 / NF
    var = (q16 - NF * mean * mean) / (NF - 1.0)
    std = jnp.sqrt(var) + 1e-8
    kl = -jnp.log(std) + (std * std + mean * mean) * 0.5 - 0.5
    kli = jnp.sum(kl[0:8]) / (8.0 * B)
    klp = jnp.sum(kl[8:16]) / (8.0 * B)

    imf = (icnt_ref[...] > 0.0).astype(jnp.float32)
    umf = (ucnt_ref[...] > 0.0).astype(jnp.float32)
    item_sum = jnp.sum(rsqi_ref[...] * imf)
    item_cnt = jnp.sum(imf)
    user_sum = jnp.sum(rsqu_ref[...] * umf)
    user_cnt = jnp.sum(umf)
    disc = item_sum / (item_cnt * float(D)) + user_sum / (user_cnt * float(D))

    inv = 1.0 / (8.0 * float(E))
    out_ref[0] = acc_ref[0] * inv
    out_ref[1] = acc_ref[1] * inv
    out_ref[2] = acc_ref[2] * inv
    out_ref[3] = disc
    out_ref[4] = kli
    out_ref[5] = klp
    out_ref[6] = 0.0
    out_ref[7] = 0.0


def _finalize(g_u, g_ip, g_in, mask3, g_u1, gbig, sv,
              rsqi, rsqu, icnt3, ucnt3):
  f32 = jnp.float32
  blk = lambda: pl.BlockSpec((CB, DP), lambda k: (k, 0))
  full = lambda *s: pl.BlockSpec(s, lambda k: tuple(0 for _ in s))
  return pl.pallas_call(
      _fin_body,
      grid=(CG,),
      in_specs=[
          blk(), blk(), blk(),
          pl.BlockSpec((1, 1, CB), lambda k: (k, 0, 0)),
          full(B, DP), full(DP, DP), full(1, DP),
          full(RG, 1, RB), full(RG, 1, RB),
          full(RG, 1, RB), full(RG, 1, RB),
      ],
      out_specs=pl.BlockSpec(memory_space=pltpu.SMEM),
      out_shape=jax.ShapeDtypeStruct((8,), f32),
      scratch_shapes=[pltpu.SMEM((4,), f32)],
      compiler_params=pltpu.CompilerParams(
          dimension_semantics=("arbitrary",)),
  )(g_u, g_ip, g_in, mask3, g_u1, gbig, sv, rsqi, rsqu, icnt3, ucnt3)


def kernel(user, item_p, item_n, mask,
           users_int_w, users_pop_w, items_int_w, items_pop_w):
  i32 = jnp.int32
  user = user.astype(i32)
  item_p = item_p.astype(i32)
  item_n = item_n.astype(i32)

  pad = ((0, 0), (0, 8 - GJ), (0, 0))
  ug_idx = jnp.pad(user.reshape(NW, GJ, 128), pad)
  pg_idx = jnp.pad(item_p.reshape(NW, GJ, 128), pad)
  ng_idx = jnp.pad(item_n.reshape(NW, GJ, 128), pad)
  u1p_idx = jnp.pad(user[:, 1].reshape(8, 1, 128), ((0, 0), (0, 7), (0, 0)))
  mask3 = mask.reshape(-1).astype(jnp.float32).reshape(CG, 1, CB)

  gbig, sv, rsqi, rsqu, tu, ti = _sweep(
      items_int_w, items_pop_w, users_int_w, users_pop_w)

  item_cnt, user_cnt, g_u, g_ip, g_in, g_u1 = _sc_gather_scatter(
      tu, ti, ug_idx, pg_idx, ng_idx, u1p_idx)

  icnt3 = item_cnt[:NV].reshape(RG, 1, RB)
  ucnt3 = user_cnt[:NV].reshape(RG, 1, RB)

  out = _finalize(g_u, g_ip, g_in, mask3, g_u1, gbig, sv,
                  rsqi, rsqu, icnt3, ucnt3)
  return (out[0], out[1], out[2], out[3], out[4], out[5])


# speedup vs baseline: 1.5073x; 1.0421x over previous
import jax
import jax.numpy as jnp
from jax import lax
from jax.experimental import pallas as pl
from jax.experimental.pallas import tpu as pltpu
from jax.experimental.pallas import tpu_sc as plsc

NV = 100000
D = 64
DP = 128
B = 1024
L = 20
E = B * L
NC, NS = 2, 16
NW = NC * NS
EPW = E // NW
GJ = EPW // 128
PRES_PAD = 100096
TILE_P = PRES_PAD // NS
RB = 5000
RG = NV // RB
CB = 4096
CG = E // CB
NF = float(NV)


def _sc_body(tu, ti, idx_all,
             item_cnt, user_cnt, g_u, g_ip, g_in, g_u1,
             pres_sp, idx_v, uidx_v, pidx_v, nidx_v, u1idx_v,
             ones_v, zer_v, r_u, r_p, r_n, sem):
  c = lax.axis_index("c")
  s = lax.axis_index("s")
  wid = c * NS + s

  def _zb(i, carry):
    zer_v[pl.ds(i * 16, 16)] = jnp.zeros((16,), jnp.float32)
    return carry
  lax.fori_loop(0, TILE_P // 16, _zb, 0)
  for i in range(8):
    ones_v[pl.ds(i * 16, 16)] = jnp.full((16,), 1.0, jnp.float32)

  pltpu.sync_copy(zer_v, pres_sp.at[pl.ds(s * TILE_P, TILE_P)])
  plsc.subcore_barrier()

  @pl.when(c == 0)
  def _():
    pltpu.sync_copy(idx_all.at[NW + 2 * s], idx_v.at[pl.ds(0, 8)])
    pltpu.sync_copy(idx_all.at[NW + 2 * s + 1], idx_v.at[pl.ds(8, 8)])
    pltpu.sync_copy(idx_all.at[2 * NW + 2 * s], idx_v.at[pl.ds(16, 8)])
    pltpu.sync_copy(idx_all.at[2 * NW + 2 * s + 1], idx_v.at[pl.ds(24, 8)])
    ds = []
    for blk in range(4):
      for j in range(GJ):
        ds.append(pltpu.async_copy(
            ones_v, pres_sp.at[idx_v.at[8 * blk + j]], sem, add=True))
    for d in ds:
      d.wait()

  @pl.when(c == 1)
  def _():
    pltpu.sync_copy(idx_all.at[2 * s], idx_v.at[pl.ds(0, 8)])
    pltpu.sync_copy(idx_all.at[2 * s + 1], idx_v.at[pl.ds(8, 8)])
    ds = []
    for blk in range(2):
      for j in range(GJ):
        ds.append(pltpu.async_copy(
            ones_v, pres_sp.at[idx_v.at[8 * blk + j]], sem, add=True))
    for d in ds:
      d.wait()

  plsc.subcore_barrier()

  pltpu.sync_copy(pres_sp.at[pl.ds(s * TILE_P, TILE_P)], zer_v)

  @pl.when(c == 0)
  def _():
    pltpu.sync_copy(zer_v, item_cnt.at[pl.ds(s * TILE_P, TILE_P)])

  @pl.when(c == 1)
  def _():
    pltpu.sync_copy(zer_v, user_cnt.at[pl.ds(s * TILE_P, TILE_P)])

  pltpu.sync_copy(idx_all.at[wid], uidx_v)
  pltpu.sync_copy(idx_all.at[NW + wid], pidx_v)
  pltpu.sync_copy(idx_all.at[2 * NW + wid], nidx_v)
  for j in range(GJ):
    d0 = pltpu.async_copy(tu.at[uidx_v.at[j]], r_u, sem)
    d1 = pltpu.async_copy(ti.at[pidx_v.at[j]], r_p, sem)
    d2 = pltpu.async_copy(ti.at[nidx_v.at[j]], r_n, sem)
    d0.wait(); d1.wait(); d2.wait()
    base = wid * EPW + j * 128
    pltpu.sync_copy(r_u, g_u.at[pl.ds(base, 128)])
    pltpu.sync_copy(r_p, g_ip.at[pl.ds(base, 128)])
    pltpu.sync_copy(r_n, g_in.at[pl.ds(base, 128)])

  @pl.when(wid < 8)
  def _():
    pltpu.sync_copy(idx_all.at[3 * NW + wid], u1idx_v)
    d = pltpu.async_copy(tu.at[u1idx_v.at[0]], r_u, sem)
    d.wait()
    pltpu.sync_copy(r_u, g_u1.at[pl.ds(wid * 128, 128)])


def _sc_gather_scatter(tu, ti, idx_all):
  f32 = jnp.float32
  out_type = [
      jax.ShapeDtypeStruct((PRES_PAD,), f32),
      jax.ShapeDtypeStruct((PRES_PAD,), f32),
      jax.ShapeDtypeStruct((E, DP), f32),
      jax.ShapeDtypeStruct((E, DP), f32),
      jax.ShapeDtypeStruct((E, DP), f32),
      jax.ShapeDtypeStruct((B, DP), f32),
  ]
  scratch = [
      pltpu.VMEM_SHARED((PRES_PAD,), f32),
      pltpu.VMEM((32, 128), jnp.int32),
      pltpu.VMEM((8, 128), jnp.int32),
      pltpu.VMEM((8, 128), jnp.int32),
      pltpu.VMEM((8, 128), jnp.int32),
      pltpu.VMEM((8, 128), jnp.int32),
      pltpu.VMEM((128,), f32),
      pltpu.VMEM((TILE_P,), f32),
      pltpu.VMEM((128, DP), f32),
      pltpu.VMEM((128, DP), f32),
      pltpu.VMEM((128, DP), f32),
      pltpu.SemaphoreType.DMA,
  ]
  fn = pl.kernel(
      _sc_body,
      out_type=out_type,
      mesh=plsc.VectorSubcoreMesh(core_axis_name="c", subcore_axis_name="s"),
      scratch_types=scratch,
      compiler_params=pltpu.CompilerParams(use_tc_tiling_on_sc=True),
  )
  return fn(tu, ti, idx_all)


def _ones8(rows, cols):
  return jnp.ones((rows, cols), jnp.float32)


def _sweep_body(ii_ref, ip_ref, ui_ref, up_ref,
                gbig_ref, sv_ref, rsqi_ref, rsqu_ref, tu_ref, ti_ref):
  k = pl.program_id(0)

  @pl.when(k == 0)
  def _():
    gbig_ref[...] = jnp.zeros((DP, DP), jnp.float32)
    sv_ref[...] = jnp.zeros((1, DP), jnp.float32)

  a = ii_ref[...]
  b = ip_ref[...]
  cu = ui_ref[...]
  du = up_ref[...]
  ti = jnp.concatenate([a, b], axis=1)
  tu = jnp.concatenate([cu, du], axis=1)
  ti_ref[...] = ti
  tu_ref[...] = tu
  gbig_ref[...] += lax.dot_general(
      ti, ti, (((0,), (0,)), ((), ())), preferred_element_type=jnp.float32)
  sv_ref[...] += jnp.sum(ti, axis=0, keepdims=True)

  on8 = _ones8(8, D)
  dnT = (((1,), (1,)), ((), ()))
  dif = a - b
  rq = lax.dot_general(on8, dif * dif, dnT,
                       preferred_element_type=jnp.float32)
  rsqi_ref[...] = rq[0:1].reshape(1, 1, RB)
  difu = cu - du
  rqu = lax.dot_general(on8, difu * difu, dnT,
                        preferred_element_type=jnp.float32)
  rsqu_ref[...] = rqu[0:1].reshape(1, 1, RB)


def _sweep(i_int_w, i_pop_w, u_int_w, u_pop_w):
  f32 = jnp.float32
  return pl.pallas_call(
      _sweep_body,
      grid=(RG,),
      in_specs=[
          pl.BlockSpec((RB, D), lambda k: (k, 0)),
          pl.BlockSpec((RB, D), lambda k: (k, 0)),
          pl.BlockSpec((RB, D), lambda k: (k, 0)),
          pl.BlockSpec((RB, D), lambda k: (k, 0)),
      ],
      out_specs=[
          pl.BlockSpec((DP, DP), lambda k: (0, 0)),
          pl.BlockSpec((1, DP), lambda k: (0, 0)),
          pl.BlockSpec((1, 1, RB), lambda k: (k, 0, 0)),
          pl.BlockSpec((1, 1, RB), lambda k: (k, 0, 0)),
          pl.BlockSpec((RB, DP), lambda k: (k, 0)),
          pl.BlockSpec((RB, DP), lambda k: (k, 0)),
      ],
      out_shape=[
          jax.ShapeDtypeStruct((DP, DP), f32),
          jax.ShapeDtypeStruct((1, DP), f32),
          jax.ShapeDtypeStruct((RG, 1, RB), f32),
          jax.ShapeDtypeStruct((RG, 1, RB), f32),
          jax.ShapeDtypeStruct((NV, DP), f32),
          jax.ShapeDtypeStruct((NV, DP), f32),
      ],
      compiler_params=pltpu.CompilerParams(
          dimension_semantics=("arbitrary",)),
  )(i_int_w, i_pop_w, u_int_w, u_pop_w)


def _nls(x):
  return jnp.maximum(-x, 0.0) + jnp.log(1.0 + jnp.exp(-jnp.abs(x)))


def _half_ones():
  r = lax.broadcasted_iota(jnp.int32, (16, DP), 0)
  cidx = lax.broadcasted_iota(jnp.int32, (16, DP), 1)
  left = jnp.logical_and(r < 8, cidx < D)
  right = jnp.logical_and(r >= 8, cidx >= D)
  return jnp.logical_or(left, right).astype(jnp.float32)


def _fin_body(gu_ref, gip_ref, gin_ref, m_ref, gu1_ref, gbig_ref, sv_ref,
              rsqi_ref, rsqu_ref, icnt_ref, ucnt_ref,
              out_ref, acc_ref):
  k = pl.program_id(0)
  dnT = (((1,), (1,)), ((), ()))

  @pl.when(k == 0)
  def _():
    acc_ref[0] = 0.0
    acc_ref[1] = 0.0
    acc_ref[2] = 0.0

  w16 = _half_ones()
  z = gu_ref[...] * (gip_ref[...] - gin_ref[...])
  xq = lax.dot_general(w16, z, dnT, preferred_element_type=jnp.float32)
  xi = xq[0:8]
  xp = xq[8:16]
  m = m_ref[0]
  acc_ref[0] += jnp.sum(m * _nls(xi))
  acc_ref[1] += jnp.sum(m * _nls(-xp) + (1.0 - m) * _nls(xp))
  acc_ref[2] += jnp.sum(_nls(xi + xp))

  @pl.when(k == CG - 1)
  def _():
    u1 = gu1_ref[...]
    rr = lax.broadcasted_iota(jnp.int32, (DP, DP), 0)
    cc = lax.broadcasted_iota(jnp.int32, (DP, DP), 1)
    bm = ((rr < D) == (cc < D)).astype(jnp.float32)
    gd = gbig_ref[...] * bm
    t = lax.dot_general(u1, gd, (((1,), (0,)), ((), ())),
                        preferred_element_type=jnp.float32)
    q16 = lax.dot_general(w16, t * u1, dnT,
                          preferred_element_type=jnp.float32)
    s16 = lax.dot_general(w16, u1 * sv_ref[...], dnT,
                          preferred_element_type=jnp.float32)
    mean = s16 / NF
    var = (q16 - NF * mean * mean) / (NF - 1.0)
    std = jnp.sqrt(var) + 1e-8
    kl = -jnp.log(std) + (std * std + mean * mean) * 0.5 - 0.5
    kli = jnp.sum(kl[0:8]) / (8.0 * B)
    klp = jnp.sum(kl[8:16]) / (8.0 * B)

    imf = (icnt_ref[...] > 0.0).astype(jnp.float32)
    umf = (ucnt_ref[...] > 0.0).astype(jnp.float32)
    item_sum = jnp.sum(rsqi_ref[...] * imf)
    item_cnt = jnp.sum(imf)
    user_sum = jnp.sum(rsqu_ref[...] * umf)
    user_cnt = jnp.sum(umf)
    disc = item_sum / (item_cnt * float(D)) + user_sum / (user_cnt * float(D))

    inv = 1.0 / (8.0 * float(E))
    out_ref[0] = acc_ref[0] * inv
    out_ref[1] = acc_ref[1] * inv
    out_ref[2] = acc_ref[2] * inv
    out_ref[3] = disc
    out_ref[4] = kli
    out_ref[5] = klp
    out_ref[6] = 0.0
    out_ref[7] = 0.0


def _finalize(g_u, g_ip, g_in, mask3, g_u1, gbig, sv,
              rsqi, rsqu, icnt3, ucnt3):
  f32 = jnp.float32
  blk = lambda: pl.BlockSpec((CB, DP), lambda k: (k, 0))
  full = lambda *s: pl.BlockSpec(s, lambda k: tuple(0 for _ in s))
  return pl.pallas_call(
      _fin_body,
      grid=(CG,),
      in_specs=[
          blk(), blk(), blk(),
          pl.BlockSpec((1, 1, CB), lambda k: (k, 0, 0)),
          full(B, DP), full(DP, DP), full(1, DP),
          full(RG, 1, RB), full(RG, 1, RB),
          full(RG, 1, RB), full(RG, 1, RB),
      ],
      out_specs=pl.BlockSpec(memory_space=pltpu.SMEM),
      out_shape=jax.ShapeDtypeStruct((8,), f32),
      scratch_shapes=[pltpu.SMEM((4,), f32)],
      compiler_params=pltpu.CompilerParams(
          dimension_semantics=("arbitrary",)),
  )(g_u, g_ip, g_in, mask3, g_u1, gbig, sv, rsqi, rsqu, icnt3, ucnt3)


def kernel(user, item_p, item_n, mask,
           users_int_w, users_pop_w, items_int_w, items_pop_w):
  i32 = jnp.int32
  user = user.astype(i32)
  item_p = item_p.astype(i32)
  item_n = item_n.astype(i32)

  u1pad = jnp.pad(user[:, 1].reshape(8, 1, 128), ((0, 0), (0, GJ - 1), (0, 0)))
  rows3 = jnp.concatenate([
      user.reshape(NW, GJ, 128), item_p.reshape(NW, GJ, 128),
      item_n.reshape(NW, GJ, 128), u1pad], axis=0)
  idx_all = jnp.pad(rows3, ((0, 0), (0, 8 - GJ), (0, 0)))
  mask3 = mask.reshape(-1).astype(jnp.float32).reshape(CG, 1, CB)

  gbig, sv, rsqi, rsqu, tu, ti = _sweep(
      items_int_w, items_pop_w, users_int_w, users_pop_w)

  item_cnt, user_cnt, g_u, g_ip, g_in, g_u1 = _sc_gather_scatter(
      tu, ti, idx_all)

  icnt3 = item_cnt[:NV].reshape(RG, 1, RB)
  ucnt3 = user_cnt[:NV].reshape(RG, 1, RB)

  out = _finalize(g_u, g_ip, g_in, mask3, g_u1, gbig, sv,
                  rsqi, rsqu, icnt3, ucnt3)
  return (out[0], out[1], out[2], out[3], out[4], out[5])


# speedup vs baseline: 1.5363x; 1.0192x over previous
import jax
import jax.numpy as jnp
from jax import lax
from jax.experimental import pallas as pl
from jax.experimental.pallas import tpu as pltpu
from jax.experimental.pallas import tpu_sc as plsc

NV = 100000
D = 64
DP = 128
B = 1024
L = 20
E = B * L
NC, NS = 2, 16
NW = NC * NS
EPW = E // NW
GJ = EPW // 128
PRES_PAD = 100096
TILE_P = PRES_PAD // NS
RB = 5000
RG = NV // RB
CB = 4096
CG = E // CB
NF = float(NV)


def _sc_body(tu, ti, idx_all,
             item_cnt, user_cnt, g_u, g_ip, g_in, g_u1,
             pres_sp, idx_v, uidx_v, pidx_v, nidx_v, u1idx_v,
             ones_v, zer_v, r_u0, r_p0, r_n0, r_u1, r_p1, r_n1,
             sem, sem_p):
  c = lax.axis_index("c")
  s = lax.axis_index("s")
  wid = c * NS + s
  bufs = ((r_u0, r_p0, r_n0), (r_u1, r_p1, r_n1))

  pltpu.sync_copy(idx_all.at[wid], uidx_v)
  pltpu.sync_copy(idx_all.at[NW + wid], pidx_v)
  pltpu.sync_copy(idx_all.at[2 * NW + wid], nidx_v)

  def _fire(j, bset):
    return (pltpu.async_copy(tu.at[uidx_v.at[j]], bset[0], sem),
            pltpu.async_copy(ti.at[pidx_v.at[j]], bset[1], sem),
            pltpu.async_copy(ti.at[nidx_v.at[j]], bset[2], sem))

  pend = _fire(0, bufs[0])

  def _zb(i, carry):
    zer_v[pl.ds(i * 16, 16)] = jnp.zeros((16,), jnp.float32)
    return carry
  lax.fori_loop(0, TILE_P // 16, _zb, 0)
  for i in range(8):
    ones_v[pl.ds(i * 16, 16)] = jnp.full((16,), 1.0, jnp.float32)

  pltpu.sync_copy(zer_v, pres_sp.at[pl.ds(s * TILE_P, TILE_P)])
  plsc.subcore_barrier()

  @pl.when(c == 0)
  def _():
    pltpu.sync_copy(idx_all.at[NW + 2 * s], idx_v.at[pl.ds(0, 8)])
    pltpu.sync_copy(idx_all.at[NW + 2 * s + 1], idx_v.at[pl.ds(8, 8)])
    pltpu.sync_copy(idx_all.at[2 * NW + 2 * s], idx_v.at[pl.ds(16, 8)])
    pltpu.sync_copy(idx_all.at[2 * NW + 2 * s + 1], idx_v.at[pl.ds(24, 8)])
    ds = []
    for blk in range(4):
      for j in range(GJ):
        ds.append(pltpu.async_copy(
            ones_v, pres_sp.at[idx_v.at[8 * blk + j]], sem_p, add=True))
    for d in ds:
      d.wait()

  @pl.when(c == 1)
  def _():
    pltpu.sync_copy(idx_all.at[2 * s], idx_v.at[pl.ds(0, 8)])
    pltpu.sync_copy(idx_all.at[2 * s + 1], idx_v.at[pl.ds(8, 8)])
    ds = []
    for blk in range(2):
      for j in range(GJ):
        ds.append(pltpu.async_copy(
            ones_v, pres_sp.at[idx_v.at[8 * blk + j]], sem_p, add=True))
    for d in ds:
      d.wait()

  plsc.subcore_barrier()

  pltpu.sync_copy(pres_sp.at[pl.ds(s * TILE_P, TILE_P)], zer_v)

  @pl.when(c == 0)
  def _():
    pltpu.sync_copy(zer_v, item_cnt.at[pl.ds(s * TILE_P, TILE_P)])

  @pl.when(c == 1)
  def _():
    pltpu.sync_copy(zer_v, user_cnt.at[pl.ds(s * TILE_P, TILE_P)])

  for j in range(GJ):
    for d in pend:
      d.wait()
    bset = bufs[j % 2]
    if j + 1 < GJ:
      pend = _fire(j + 1, bufs[(j + 1) % 2])
    base = wid * EPW + j * 128
    pltpu.sync_copy(bset[0], g_u.at[pl.ds(base, 128)])
    pltpu.sync_copy(bset[1], g_ip.at[pl.ds(base, 128)])
    pltpu.sync_copy(bset[2], g_in.at[pl.ds(base, 128)])

  @pl.when(wid < 8)
  def _():
    pltpu.sync_copy(idx_all.at[3 * NW + wid], u1idx_v)
    d = pltpu.async_copy(tu.at[u1idx_v.at[0]], r_u0, sem)
    d.wait()
    pltpu.sync_copy(r_u0, g_u1.at[pl.ds(wid * 128, 128)])


def _sc_gather_scatter(tu, ti, idx_all):
  f32 = jnp.float32
  out_type = [
      jax.ShapeDtypeStruct((PRES_PAD,), f32),
      jax.ShapeDtypeStruct((PRES_PAD,), f32),
      jax.ShapeDtypeStruct((E, DP), f32),
      jax.ShapeDtypeStruct((E, DP), f32),
      jax.ShapeDtypeStruct((E, DP), f32),
      jax.ShapeDtypeStruct((B, DP), f32),
  ]
  scratch = [
      pltpu.VMEM_SHARED((PRES_PAD,), f32),
      pltpu.VMEM((32, 128), jnp.int32),
      pltpu.VMEM((8, 128), jnp.int32),
      pltpu.VMEM((8, 128), jnp.int32),
      pltpu.VMEM((8, 128), jnp.int32),
      pltpu.VMEM((8, 128), jnp.int32),
      pltpu.VMEM((128,), f32),
      pltpu.VMEM((TILE_P,), f32),
      pltpu.VMEM((128, DP), f32),
      pltpu.VMEM((128, DP), f32),
      pltpu.VMEM((128, DP), f32),
      pltpu.VMEM((128, DP), f32),
      pltpu.VMEM((128, DP), f32),
      pltpu.VMEM((128, DP), f32),
      pltpu.SemaphoreType.DMA,
      pltpu.SemaphoreType.DMA,
  ]
  fn = pl.kernel(
      _sc_body,
      out_type=out_type,
      mesh=plsc.VectorSubcoreMesh(core_axis_name="c", subcore_axis_name="s"),
      scratch_types=scratch,
      compiler_params=pltpu.CompilerParams(use_tc_tiling_on_sc=True),
  )
  return fn(tu, ti, idx_all)


def _ones8(rows, cols):
  return jnp.ones((rows, cols), jnp.float32)


def _sweep_body(ii_ref, ip_ref, ui_ref, up_ref,
                gbig_ref, sv_ref, rsqi_ref, rsqu_ref, tu_ref, ti_ref):
  k = pl.program_id(0)

  @pl.when(k == 0)
  def _():
    gbig_ref[...] = jnp.zeros((DP, DP), jnp.float32)
    sv_ref[...] = jnp.zeros((1, DP), jnp.float32)

  a = ii_ref[...]
  b = ip_ref[...]
  cu = ui_ref[...]
  du = up_ref[...]
  ti = jnp.concatenate([a, b], axis=1)
  tu = jnp.concatenate([cu, du], axis=1)
  ti_ref[...] = ti
  tu_ref[...] = tu
  gbig_ref[...] += lax.dot_general(
      ti, ti, (((0,), (0,)), ((), ())), preferred_element_type=jnp.float32)
  sv_ref[...] += jnp.sum(ti, axis=0, keepdims=True)

  on8 = _ones8(8, D)
  dnT = (((1,), (1,)), ((), ()))
  dif = a - b
  rq = lax.dot_general(on8, dif * dif, dnT,
                       preferred_element_type=jnp.float32)
  rsqi_ref[...] = rq[0:1].reshape(1, 1, RB)
  difu = cu - du
  rqu = lax.dot_general(on8, difu * difu, dnT,
                        preferred_element_type=jnp.float32)
  rsqu_ref[...] = rqu[0:1].reshape(1, 1, RB)


def _sweep(i_int_w, i_pop_w, u_int_w, u_pop_w):
  f32 = jnp.float32
  return pl.pallas_call(
      _sweep_body,
      grid=(RG,),
      in_specs=[
          pl.BlockSpec((RB, D), lambda k: (k, 0)),
          pl.BlockSpec((RB, D), lambda k: (k, 0)),
          pl.BlockSpec((RB, D), lambda k: (k, 0)),
          pl.BlockSpec((RB, D), lambda k: (k, 0)),
      ],
      out_specs=[
          pl.BlockSpec((DP, DP), lambda k: (0, 0)),
          pl.BlockSpec((1, DP), lambda k: (0, 0)),
          pl.BlockSpec((1, 1, RB), lambda k: (k, 0, 0)),
          pl.BlockSpec((1, 1, RB), lambda k: (k, 0, 0)),
          pl.BlockSpec((RB, DP), lambda k: (k, 0)),
          pl.BlockSpec((RB, DP), lambda k: (k, 0)),
      ],
      out_shape=[
          jax.ShapeDtypeStruct((DP, DP), f32),
          jax.ShapeDtypeStruct((1, DP), f32),
          jax.ShapeDtypeStruct((RG, 1, RB), f32),
          jax.ShapeDtypeStruct((RG, 1, RB), f32),
          jax.ShapeDtypeStruct((NV, DP), f32),
          jax.ShapeDtypeStruct((NV, DP), f32),
      ],
      compiler_params=pltpu.CompilerParams(
          dimension_semantics=("arbitrary",)),
  )(i_int_w, i_pop_w, u_int_w, u_pop_w)


def _nls(x):
  return jnp.maximum(-x, 0.0) + jnp.log(1.0 + jnp.exp(-jnp.abs(x)))


def _half_ones():
  r = lax.broadcasted_iota(jnp.int32, (16, DP), 0)
  cidx = lax.broadcasted_iota(jnp.int32, (16, DP), 1)
  left = jnp.logical_and(r < 8, cidx < D)
  right = jnp.logical_and(r >= 8, cidx >= D)
  return jnp.logical_or(left, right).astype(jnp.float32)


def _fin_body(gu_ref, gip_ref, gin_ref, m_ref, gu1_ref, gbig_ref, sv_ref,
              rsqi_ref, rsqu_ref, icnt_ref, ucnt_ref,
              out_ref, acc_ref):
  k = pl.program_id(0)
  dnT = (((1,), (1,)), ((), ()))

  @pl.when(k == 0)
  def _():
    acc_ref[0] = 0.0
    acc_ref[1] = 0.0
    acc_ref[2] = 0.0

  w16 = _half_ones()
  z = gu_ref[...] * (gip_ref[...] - gin_ref[...])
  xq = lax.dot_general(w16, z, dnT, preferred_element_type=jnp.float32)
  xi = xq[0:8]
  xp = xq[8:16]
  m = m_ref[0]
  acc_ref[0] += jnp.sum(m * _nls(xi))
  acc_ref[1] += jnp.sum(m * _nls(-xp) + (1.0 - m) * _nls(xp))
  acc_ref[2] += jnp.sum(_nls(xi + xp))

  @pl.when(k == CG - 1)
  def _():
    u1 = gu1_ref[...]
    rr = lax.broadcasted_iota(jnp.int32, (DP, DP), 0)
    cc = lax.broadcasted_iota(jnp.int32, (DP, DP), 1)
    bm = ((rr < D) == (cc < D)).astype(jnp.float32)
    gd = gbig_ref[...] * bm
    t = lax.dot_general(u1, gd, (((1,), (0,)), ((), ())),
                        preferred_element_type=jnp.float32)
    q16 = lax.dot_general(w16, t * u1, dnT,
                          preferred_element_type=jnp.float32)
    s16 = lax.dot_general(w16, u1 * sv_ref[...], dnT,
                          preferred_element_type=jnp.float32)
    mean = s16 / NF
    var = (q16 - NF * mean * mean) / (NF - 1.0)
    std = jnp.sqrt(var) + 1e-8
    kl = -jnp.log(std) + (std * std + mean * mean) * 0.5 - 0.5
    kli = jnp.sum(kl[0:8]) / (8.0 * B)
    klp = jnp.sum(kl[8:16]) / (8.0 * B)

    imf = (icnt_ref[...] > 0.0).astype(jnp.float32)
    umf = (ucnt_ref[...] > 0.0).astype(jnp.float32)
    item_sum = jnp.sum(rsqi_ref[...] * imf)
    item_cnt = jnp.sum(imf)
    user_sum = jnp.sum(rsqu_ref[...] * umf)
    user_cnt = jnp.sum(umf)
    disc = item_sum / (item_cnt * float(D)) + user_sum / (user_cnt * float(D))

    inv = 1.0 / (8.0 * float(E))
    out_ref[0] = acc_ref[0] * inv
    out_ref[1] = acc_ref[1] * inv
    out_ref[2] = acc_ref[2] * inv
    out_ref[3] = disc
    out_ref[4] = kli
    out_ref[5] = klp
    out_ref[6] = 0.0
    out_ref[7] = 0.0


def _finalize(g_u, g_ip, g_in, mask3, g_u1, gbig, sv,
              rsqi, rsqu, icnt3, ucnt3):
  f32 = jnp.float32
  blk = lambda: pl.BlockSpec((CB, DP), lambda k: (k, 0))
  full = lambda *s: pl.BlockSpec(s, lambda k: tuple(0 for _ in s))
  return pl.pallas_call(
      _fin_body,
      grid=(CG,),
      in_specs=[
          blk(), blk(), blk(),
          pl.BlockSpec((1, 1, CB), lambda k: (k, 0, 0)),
          full(B, DP), full(DP, DP), full(1, DP),
          full(RG, 1, RB), full(RG, 1, RB),
          full(RG, 1, RB), full(RG, 1, RB),
      ],
      out_specs=pl.BlockSpec(memory_space=pltpu.SMEM),
      out_shape=jax.ShapeDtypeStruct((8,), f32),
      scratch_shapes=[pltpu.SMEM((4,), f32)],
      compiler_params=pltpu.CompilerParams(
          dimension_semantics=("arbitrary",)),
  )(g_u, g_ip, g_in, mask3, g_u1, gbig, sv, rsqi, rsqu, icnt3, ucnt3)


def kernel(user, item_p, item_n, mask,
           users_int_w, users_pop_w, items_int_w, items_pop_w):
  i32 = jnp.int32
  user = user.astype(i32)
  item_p = item_p.astype(i32)
  item_n = item_n.astype(i32)

  u1pad = jnp.pad(user[:, 1].reshape(8, 1, 128), ((0, 0), (0, GJ - 1), (0, 0)))
  rows3 = jnp.concatenate([
      user.reshape(NW, GJ, 128), item_p.reshape(NW, GJ, 128),
      item_n.reshape(NW, GJ, 128), u1pad], axis=0)
  idx_all = jnp.pad(rows3, ((0, 0), (0, 8 - GJ), (0, 0)))
  mask3 = mask.reshape(-1).astype(jnp.float32).reshape(CG, 1, CB)

  gbig, sv, rsqi, rsqu, tu, ti = _sweep(
      items_int_w, items_pop_w, users_int_w, users_pop_w)

  item_cnt, user_cnt, g_u, g_ip, g_in, g_u1 = _sc_gather_scatter(
      tu, ti, idx_all)

  icnt3 = item_cnt[:NV].reshape(RG, 1, RB)
  ucnt3 = user_cnt[:NV].reshape(RG, 1, RB)

  out = _finalize(g_u, g_ip, g_in, mask3, g_u1, gbig, sv,
                  rsqi, rsqu, icnt3, ucnt3)
  return (out[0], out[1], out[2], out[3], out[4], out[5])
